# baseline (speedup 1.0000x reference)
import functools

import jax
import jax.numpy as jnp
from jax import lax
from jax.experimental import pallas as pl
from jax.experimental.pallas import tpu as pltpu
from jax.experimental.pallas import tpu_sc as plsc

VOCAB_ROWS = 100000
EMB = 64
B = 4096
QLEN = 20
NDOCS = 8
DLEN = 50
EPS = 1e-8

NC = 2
NS = 16
NW = NC * NS

Q_SEGS_W = B // NW

D_SEGS_W = B * NDOCS // NW


def _rsqrt(x):
    y = plsc.bitcast(
        jnp.full((16,), 0x5F3759DF, jnp.int32)
        - lax.shift_right_logical(plsc.bitcast(x, jnp.int32), 1),
        jnp.float32)
    xh = x * 0.5
    for _ in range(4):
        y = y * (1.5 - xh * y * y)
    return y


def _sc_scores(table, qidx, didx):
    mesh = plsc.VectorSubcoreMesh(
        core_axis_name="c", subcore_axis_name="s",
        num_cores=NC, num_subcores=NS)

    NBUF_D = 8
    NBUF_Q = 4
    Q_GROUPS = Q_SEGS_W // NBUF_Q

    @functools.partial(
        pl.kernel,
        out_type=jax.ShapeDtypeStruct((B * NDOCS,), jnp.float32),
        mesh=mesh,
        compiler_params=pltpu.CompilerParams(
            use_tc_tiling_on_sc=False, needs_layout_passes=False),
        scratch_types=[
            pltpu.VMEM((Q_SEGS_W, 128), jnp.int32),
            pltpu.VMEM((Q_SEGS_W // 2, NDOCS, 128), jnp.int32),
            [pltpu.VMEM((24, EMB), jnp.bfloat16)] * NBUF_Q,
            [pltpu.VMEM((56, EMB), jnp.bfloat16)] * NBUF_D,
            pltpu.VMEM((Q_SEGS_W, EMB), jnp.float32),
            pltpu.VMEM((Q_SEGS_W * NDOCS,), jnp.float32),
            [pltpu.SemaphoreType.DMA] * NBUF_Q,
            [pltpu.SemaphoreType.DMA] * NBUF_D,
        ],
    )
    def pool(table_hbm, qidx_hbm, didx_hbm, scores_hbm,
             qidx_v, didx_v, qbufs, dbufs, qstage_v, sstage_v, qsems, dsems):
        wid = lax.axis_index("s") * NC + lax.axis_index("c")

        pltpu.sync_copy(qidx_hbm.at[pl.ds(wid * Q_SEGS_W, Q_SEGS_W)], qidx_v)

        def q_issue(c, b):
            idx = qidx_v.at[c, pl.ds(0, 24)]
            pltpu.async_copy(table_hbm.at[idx], qbufs[b], qsems[b])

        def q_wait(b):
            pltpu.make_async_copy(
                table_hbm.at[qidx_v.at[0, pl.ds(0, 24)]],
                qbufs[b], qsems[b]).wait()

        def d_issue(c, b):
            idx = didx_v.at[c // NDOCS, c % NDOCS, pl.ds(0, 56)]
            pltpu.async_copy(table_hbm.at[idx], dbufs[b], dsems[b])

        def d_wait(b):
            pltpu.make_async_copy(
                table_hbm.at[didx_v.at[0, 0, pl.ds(0, 56)]],
                dbufs[b], dsems[b]).wait()

        def reduce_seg(buf, row0, seg_len):
            half = seg_len // 2

            def load2(r, lanes):
                return plsc.unpack(buf[r, lanes],
                                   format=plsc.PackFormat.INTERLEAVED)

            sums = []
            for j in range(EMB // 32):
                lanes = pl.ds(j * 32, 32)
                a0, a1 = load2(row0, lanes)
                b0, b1 = load2(row0 + half, lanes)
                for r in range(1, half):
                    u0, u1 = load2(row0 + r, lanes)
                    a0, a1 = a0 + u0, a1 + u1
                    v0, v1 = load2(row0 + half + r, lanes)
                    b0, b1 = b0 + v0, b1 + v1
                sums.append(a0 + b0)
                sums.append(a1 + b1)
            return sums

        for b in range(NBUF_Q):
            q_issue(b, b)

        def q_group(g, carry):
            for b in range(NBUF_Q):
                c = g * NBUF_Q + b
                q_wait(b)
                sums = reduce_seg(qbufs[b], 0, QLEN)
                for j in range(4):
                    qstage_v[c, pl.ds(j * 16, 16)] = sums[j]
                nxt = c + NBUF_Q
                @pl.when(nxt < Q_SEGS_W)
                def _():
                    q_issue(nxt, b)
            return carry
        lax.fori_loop(0, Q_GROUPS, q_group, 0)

        lanes16 = lax.broadcasted_iota(jnp.int32, (16,), 0)
        HALF_Q = Q_SEGS_W // 2
        HALF_CHUNKS = HALF_Q * NDOCS

        for half in range(2):
            pltpu.sync_copy(
                didx_hbm.at[pl.ds(wid * Q_SEGS_W + half * HALF_Q, HALF_Q)],
                didx_v)
            for b in range(NBUF_D):
                d_issue(b, b)

            def d_group(lq, carry, half=half):
                q = half * HALF_Q + lq
                qs = [qstage_v[q, pl.ds(j * 16, 16)] for j in range(4)]
                qn2 = jnp.sum(qs[0] * qs[0] + qs[1] * qs[1]
                              + qs[2] * qs[2] + qs[3] * qs[3])
                qinv = _rsqrt(jnp.maximum(jnp.full((16,), qn2, jnp.float32),
                                          (QLEN * EPS) ** 2))
                dotv = jnp.zeros((16,), jnp.float32)
                dn2v = jnp.zeros((16,), jnp.float32)
                for b in range(NBUF_D):
                    c = lq * NBUF_D + b
                    d_wait(b)
                    ds_ = reduce_seg(dbufs[b], 0, DLEN)
                    dot = jnp.sum(qs[0] * ds_[0] + qs[1] * ds_[1]
                                  + qs[2] * ds_[2] + qs[3] * ds_[3])
                    dn2 = jnp.sum(ds_[0] * ds_[0] + ds_[1] * ds_[1]
                                  + ds_[2] * ds_[2] + ds_[3] * ds_[3])
                    dotv = jnp.where(lanes16 == b, dot, dotv)
                    dn2v = jnp.where(lanes16 == b, dn2, dn2v)
                    nxt = c + NBUF_D
                    @pl.when(nxt < HALF_CHUNKS)
                    def _():
                        d_issue(nxt, b)
                dinv = _rsqrt(jnp.maximum(dn2v, (DLEN * EPS) ** 2))
                score = dotv * qinv * dinv
                plsc.store_scatter(sstage_v, [q * NDOCS + lanes16], score,
                                   mask=lanes16 < NDOCS)
                return carry
            lax.fori_loop(0, HALF_Q, d_group, 0)

        pltpu.sync_copy(
            sstage_v,
            scores_hbm.at[pl.ds(wid * Q_SEGS_W * NDOCS, Q_SEGS_W * NDOCS)])

    return pool(table, qidx, didx)


def kernel(batch_queries, query_len, batch_docs, doc_len, W):
    del query_len, doc_len
    W = W.astype(jnp.bfloat16)
    qidx = jnp.pad(batch_queries.astype(jnp.int32),
                   ((0, 0), (0, 128 - QLEN)))
    didx = jnp.pad(batch_docs.astype(jnp.int32),
                   ((0, 0), (0, 0), (0, 128 - DLEN)))
    return _sc_scores(W, qidx, didx).reshape(B, NDOCS)


# speedup vs baseline: 8.1634x; 8.1634x over previous
import functools

import jax
import jax.numpy as jnp
from jax import lax
from jax.experimental import pallas as pl
from jax.experimental.pallas import tpu as pltpu
from jax.experimental.pallas import tpu_sc as plsc

VOCAB_ROWS = 100000
EMB = 64
B = 4096
QLEN = 20
NDOCS = 8
DLEN = 50
EPS = 1e-8

NC = 2
NS = 16
NW = NC * NS

Q_SEGS_W = B // NW

D_SEGS_W = B * NDOCS // NW


def _rsqrt(x):
    y = plsc.bitcast(
        jnp.full((16,), 0x5F3759DF, jnp.int32)
        - lax.shift_right_logical(plsc.bitcast(x, jnp.int32), 1),
        jnp.float32)
    xh = x * 0.5
    for _ in range(4):
        y = y * (1.5 - xh * y * y)
    return y


def _sc_scores(table, qidx, didx):
    mesh = plsc.VectorSubcoreMesh(
        core_axis_name="c", subcore_axis_name="s",
        num_cores=NC, num_subcores=NS)

    NBUF_D = 8
    NBUF_Q = 4
    Q_GROUPS = Q_SEGS_W // NBUF_Q

    @functools.partial(
        pl.kernel,
        out_type=jax.ShapeDtypeStruct((B * NDOCS,), jnp.float32),
        mesh=mesh,
        compiler_params=pltpu.CompilerParams(
            use_tc_tiling_on_sc=False, needs_layout_passes=False),
        scratch_types=[
            pltpu.VMEM((Q_SEGS_W, QLEN), jnp.int32),
            pltpu.VMEM((Q_SEGS_W, NDOCS, DLEN), jnp.int32),
            [pltpu.VMEM((QLEN, EMB), jnp.bfloat16)] * NBUF_Q,
            [pltpu.VMEM((DLEN, EMB), jnp.bfloat16)] * NBUF_D,
            pltpu.VMEM((Q_SEGS_W, EMB), jnp.float32),
            pltpu.VMEM((Q_SEGS_W * NDOCS,), jnp.float32),
            [pltpu.SemaphoreType.DMA] * NBUF_Q,
            [pltpu.SemaphoreType.DMA] * NBUF_D,
        ],
    )
    def pool(table_hbm, qidx_hbm, didx_hbm, scores_hbm,
             qidx_v, didx_v, qbufs, dbufs, qstage_v, sstage_v, qsems, dsems):
        wid = lax.axis_index("s") * NC + lax.axis_index("c")

        pltpu.sync_copy(qidx_hbm.at[pl.ds(wid * Q_SEGS_W, Q_SEGS_W)], qidx_v)
        pltpu.sync_copy(didx_hbm.at[pl.ds(wid * Q_SEGS_W, Q_SEGS_W)], didx_v)

        def q_issue(c, b):
            idx = qidx_v.at[c]
            pltpu.async_copy(table_hbm.at[idx], qbufs[b], qsems[b])

        def q_wait(b):
            pltpu.make_async_copy(
                table_hbm.at[qidx_v.at[0]],
                qbufs[b], qsems[b]).wait()

        def d_issue(c, b):
            idx = didx_v.at[c // NDOCS, c % NDOCS]
            pltpu.async_copy(table_hbm.at[idx], dbufs[b], dsems[b])

        def d_wait(b):
            pltpu.make_async_copy(
                table_hbm.at[didx_v.at[0, 0]],
                dbufs[b], dsems[b]).wait()

        def reduce_seg(buf, row0, seg_len):
            pairs = seg_len // 2
            hp = pairs // 2

            sums = []
            for j in range(EMB // 32):
                lanes = pl.ds(j * 32, 32)

                def pair(p):
                    t = buf[row0 + 2 * p, lanes] + buf[row0 + 2 * p + 1,
                                                       lanes]
                    return plsc.unpack(t, format=plsc.PackFormat.INTERLEAVED)

                a0, a1 = pair(0)
                b0, b1 = pair(hp)
                for p in range(1, hp):
                    u0, u1 = pair(p)
                    a0, a1 = a0 + u0, a1 + u1
                    v0, v1 = pair(hp + p)
                    b0, b1 = b0 + v0, b1 + v1
                for p in range(2 * hp, pairs):
                    u0, u1 = pair(p)
                    a0, a1 = a0 + u0, a1 + u1
                sums.append(a0 + b0)
                sums.append(a1 + b1)
            return sums

        for b in range(NBUF_Q):
            q_issue(b, b)

        def q_group(g, carry):
            for b in range(NBUF_Q):
                c = g * NBUF_Q + b
                q_wait(b)
                sums = reduce_seg(qbufs[b], 0, QLEN)
                for j in range(4):
                    qstage_v[c, pl.ds(j * 16, 16)] = sums[j]
                nxt = c + NBUF_Q
                @pl.when(nxt < Q_SEGS_W)
                def _():
                    q_issue(nxt, b)
            return carry
        lax.fori_loop(0, Q_GROUPS, q_group, 0)

        for b in range(NBUF_D):
            d_issue(b, b)

        lanes16 = lax.broadcasted_iota(jnp.int32, (16,), 0)

        def d_group(q, carry):
            qs = [qstage_v[q, pl.ds(j * 16, 16)] for j in range(4)]
            qn2 = jnp.sum(qs[0] * qs[0] + qs[1] * qs[1]
                          + qs[2] * qs[2] + qs[3] * qs[3])
            qinv = _rsqrt(jnp.maximum(jnp.full((16,), qn2, jnp.float32),
                                      (QLEN * EPS) ** 2))
            dotv = jnp.zeros((16,), jnp.float32)
            dn2v = jnp.zeros((16,), jnp.float32)
            for b in range(NBUF_D):
                c = q * NBUF_D + b
                d_wait(b)
                ds_ = reduce_seg(dbufs[b], 0, DLEN)
                dot = jnp.sum(qs[0] * ds_[0] + qs[1] * ds_[1]
                              + qs[2] * ds_[2] + qs[3] * ds_[3])
                dn2 = jnp.sum(ds_[0] * ds_[0] + ds_[1] * ds_[1]
                              + ds_[2] * ds_[2] + ds_[3] * ds_[3])
                dotv = jnp.where(lanes16 == b, dot, dotv)
                dn2v = jnp.where(lanes16 == b, dn2, dn2v)
                nxt = c + NBUF_D
                @pl.when(nxt < D_SEGS_W)
                def _():
                    d_issue(nxt, b)
            dinv = _rsqrt(jnp.maximum(dn2v, (DLEN * EPS) ** 2))
            score = dotv * qinv * dinv
            plsc.store_scatter(sstage_v, [q * NDOCS + lanes16], score,
                               mask=lanes16 < NDOCS)
            return carry
        lax.fori_loop(0, Q_SEGS_W, d_group, 0)

        pltpu.sync_copy(
            sstage_v,
            scores_hbm.at[pl.ds(wid * Q_SEGS_W * NDOCS, Q_SEGS_W * NDOCS)])

    return pool(table, qidx, didx)


def kernel(batch_queries, query_len, batch_docs, doc_len, W):
    del query_len, doc_len
    W = W.astype(jnp.bfloat16)
    qidx = batch_queries.astype(jnp.int32)
    didx = batch_docs.astype(jnp.int32)
    return _sc_scores(W, qidx, didx).reshape(B, NDOCS)
